# baseline (speedup 1.0000x reference)
import functools

import jax
import jax.numpy as jnp
from jax import lax
from jax.experimental import pallas as pl
from jax.experimental.pallas import tpu as pltpu
from jax.experimental.pallas import tpu_sc as plsc

E = 16
K = 2
BLK = 256
NP = 8192
P = NP + E * BLK
NB = P // BLK

NC = 2
NS = 16
NW = NC * NS
PER_W = NP // NW
CHUNK = 32
NPAIR = PER_W // (2 * CHUNK)


def _cumsum_axis1(x):
    n = x.shape[1]
    s = 1
    while s < n:
        x = x + jnp.concatenate(
            [jnp.zeros((x.shape[0], s), x.dtype), x[:, :-s]], axis=1)
        s *= 2
    return x


def _routing_body(idx_ref, dest_ref, be_ref, tb_ref):
    idx = idx_ref[...]
    e_iota = lax.broadcasted_iota(jnp.int32, (E, NP), 0)
    oh = (jnp.broadcast_to(idx, (E, NP)) == e_iota).astype(jnp.int32)
    cum = _cumsum_axis1(oh)
    rank = jnp.sum(jnp.where(oh == 1, cum - 1, 0), axis=0, keepdims=True)
    counts = cum[:, NP - 1:NP]
    nbk = (counts + (BLK - 1)) // BLK
    cc = nbk
    s = 1
    while s < E:
        cc = cc + jnp.concatenate(
            [jnp.zeros((s, 1), jnp.int32), cc[:-s, :]], axis=0)
        s *= 2
    blk_off = cc - nbk
    tb_ref[...] = cc[E - 1:E, :]
    dest_ref[...] = jnp.sum(
        jnp.where(oh == 1, jnp.broadcast_to(blk_off * BLK, (E, NP)), 0),
        axis=0, keepdims=True) + rank
    b_iota = lax.broadcasted_iota(jnp.int32, (E, NB), 1)
    cmp = (b_iota >= jnp.broadcast_to(blk_off, (E, NB))).astype(jnp.int32)
    be = jnp.sum(cmp, axis=0, keepdims=True) - 1
    be_ref[...] = jnp.clip(be, 0, E - 1)


def _routing(idx2):
    return pl.pallas_call(
        _routing_body,
        out_shape=(
            jax.ShapeDtypeStruct((1, NP), jnp.int32),
            jax.ShapeDtypeStruct((1, NB), jnp.int32),
            jax.ShapeDtypeStruct((1, 1), jnp.int32),
        ),
    )(idx2)


NCHUNK = PER_W // CHUNK


def _dispatch_body(x_hbm, dest_hbm, xs_hbm, didx_all, sidx_all,
                   rows0, rows1, sg0, sg1, ss0, ss1, si):
    wid = lax.axis_index("s") * NC + lax.axis_index("c")
    base_w = wid * PER_W

    for i in range(NCHUNK):
        pltpu.async_copy(
            dest_hbm.at[pl.ds(base_w + i * CHUNK, CHUNK)], didx_all.at[i], si)
    for i in range(NCHUNK):
        for v in range(CHUNK // 16):
            vec = jnp.bitwise_and(
                base_w + i * CHUNK + v * 16
                + lax.broadcasted_iota(jnp.int32, (16,), 0),
                NP // K - 1)
            sidx_all[i, pl.ds(v * 16, 16)] = vec
    for i in range(NCHUNK):
        pltpu.make_async_copy(
            dest_hbm.at[pl.ds(base_w + i * CHUNK, CHUNK)], didx_all.at[i],
            si).wait()

    def pair(g, carry):
        i0 = g * 2

        @pl.when(g > 0)
        def _():
            pltpu.make_async_copy(
                rows0, xs_hbm.at[didx_all.at[i0 - 2]], ss0).wait()
            pltpu.make_async_copy(
                rows1, xs_hbm.at[didx_all.at[i0 - 1]], ss1).wait()

        pltpu.async_copy(x_hbm.at[sidx_all.at[i0]], rows0, sg0)
        pltpu.async_copy(x_hbm.at[sidx_all.at[i0 + 1]], rows1, sg1)
        pltpu.make_async_copy(x_hbm.at[sidx_all.at[i0]], rows0, sg0).wait()
        pltpu.async_copy(rows0, xs_hbm.at[didx_all.at[i0]], ss0)
        pltpu.make_async_copy(x_hbm.at[sidx_all.at[i0 + 1]], rows1, sg1).wait()
        pltpu.async_copy(rows1, xs_hbm.at[didx_all.at[i0 + 1]], ss1)
        return carry

    lax.fori_loop(0, NPAIR, pair, 0)
    pltpu.make_async_copy(
        rows0, xs_hbm.at[didx_all.at[NCHUNK - 2]], ss0).wait()
    pltpu.make_async_copy(
        rows1, xs_hbm.at[didx_all.at[NCHUNK - 1]], ss1).wait()


def _dispatch(x, dest):
    D = x.shape[1]
    mesh = plsc.VectorSubcoreMesh(core_axis_name="c", subcore_axis_name="s")
    fn = pl.kernel(
        _dispatch_body,
        out_type=jax.ShapeDtypeStruct((P, D), jnp.float32),
        mesh=mesh,
        scratch_types=[
            pltpu.VMEM((NCHUNK, CHUNK), jnp.int32),
            pltpu.VMEM((NCHUNK, CHUNK), jnp.int32),
            pltpu.VMEM((CHUNK, D), jnp.float32),
            pltpu.VMEM((CHUNK, D), jnp.float32),
            pltpu.SemaphoreType.DMA,
            pltpu.SemaphoreType.DMA,
            pltpu.SemaphoreType.DMA,
            pltpu.SemaphoreType.DMA,
            pltpu.SemaphoreType.DMA,
        ],
    )
    return fn(x, dest)


def _matmul_body(be_ref, tb_ref, x_ref, w_ref, b_ref, o_ref):
    blk = pl.program_id(0)

    @pl.when(blk < tb_ref[0])
    def _():
        acc = lax.dot_general(
            x_ref[...], w_ref[0],
            dimension_numbers=(((1,), (1,)), ((), ())),
            preferred_element_type=jnp.float32)
        o_ref[...] = acc + b_ref[0]


def _matmul(xs, W, bias, be, tb):
    D = xs.shape[1]
    grid_spec = pltpu.PrefetchScalarGridSpec(
        num_scalar_prefetch=2,
        grid=(NB,),
        in_specs=[
            pl.BlockSpec((BLK, D),
                         lambda b, be, tb: (jnp.where(b < tb[0], b, 0), 0)),
            pl.BlockSpec((1, D, D), lambda b, be, tb: (be[b], 0, 0)),
            pl.BlockSpec((1, 1, D), lambda b, be, tb: (be[b], 0, 0)),
        ],
        out_specs=pl.BlockSpec(
            (BLK, D), lambda b, be, tb: (jnp.where(b < tb[0], b, NB - 1), 0)),
    )
    return pl.pallas_call(
        _matmul_body,
        grid_spec=grid_spec,
        out_shape=jax.ShapeDtypeStruct((P, D), jnp.float32),
    )(be, tb, xs, W, bias.reshape(E, 1, D))


def _combine_body(ys_hbm, dest_hbm, out_hbm, didx_all, rows0, rows1,
                  sg0, sg1, si):
    wid = lax.axis_index("s") * NC + lax.axis_index("c")
    kk = wid // (NW // K)
    n0 = (wid % (NW // K)) * PER_W
    bb = n0 // (NP // K // 2)
    s_base = n0 % (NP // K // 2)

    p_start = kk * (NP // K) + n0

    for i in range(NCHUNK):
        pltpu.async_copy(
            dest_hbm.at[pl.ds(p_start + i * CHUNK, CHUNK)], didx_all.at[i], si)
    for i in range(NCHUNK):
        pltpu.make_async_copy(
            dest_hbm.at[pl.ds(p_start + i * CHUNK, CHUNK)], didx_all.at[i],
            si).wait()

    def pair(g, carry):
        i0 = g * 2
        s0 = s_base + i0 * CHUNK

        @pl.when(g > 0)
        def _():
            pltpu.make_async_copy(
                rows0, out_hbm.at[bb, pl.ds(s0 - 2 * CHUNK, CHUNK), kk], sg0).wait()
            pltpu.make_async_copy(
                rows1, out_hbm.at[bb, pl.ds(s0 - CHUNK, CHUNK), kk], sg1).wait()

        pltpu.async_copy(ys_hbm.at[didx_all.at[i0]], rows0, sg0)
        pltpu.async_copy(ys_hbm.at[didx_all.at[i0 + 1]], rows1, sg1)
        pltpu.make_async_copy(ys_hbm.at[didx_all.at[i0]], rows0, sg0).wait()
        pltpu.async_copy(rows0, out_hbm.at[bb, pl.ds(s0, CHUNK), kk], sg0)
        pltpu.make_async_copy(ys_hbm.at[didx_all.at[i0 + 1]], rows1, sg1).wait()
        pltpu.async_copy(rows1, out_hbm.at[bb, pl.ds(s0 + CHUNK, CHUNK), kk], sg1)
        return carry

    lax.fori_loop(0, NPAIR, pair, 0)
    s_last = s_base + (NPAIR * 2 - 2) * CHUNK
    pltpu.make_async_copy(
        rows0, out_hbm.at[bb, pl.ds(s_last, CHUNK), kk], sg0).wait()
    pltpu.make_async_copy(
        rows1, out_hbm.at[bb, pl.ds(s_last + CHUNK, CHUNK), kk], sg1).wait()


def _combine(ys, dest, Bb, Ss):
    D = ys.shape[1]
    mesh = plsc.VectorSubcoreMesh(core_axis_name="c", subcore_axis_name="s")
    fn = pl.kernel(
        _combine_body,
        out_type=jax.ShapeDtypeStruct((Bb, Ss, K, D), jnp.float32),
        mesh=mesh,
        scratch_types=[
            pltpu.VMEM((NCHUNK, CHUNK), jnp.int32),
            pltpu.VMEM((CHUNK, D), jnp.float32),
            pltpu.VMEM((CHUNK, D), jnp.float32),
            pltpu.SemaphoreType.DMA,
            pltpu.SemaphoreType.DMA,
            pltpu.SemaphoreType.DMA,
        ],
    )
    return fn(ys, dest)


def kernel(inputs, expert_indices, W, bias):
    Bb, Ss, Dd = inputs.shape
    x = inputs.reshape(Bb * Ss, Dd)
    idx2 = (expert_indices.reshape(Bb * Ss, K).T
            .reshape(1, NP).astype(jnp.int32))
    dest2, be2, tb2 = _routing(idx2)
    dest = dest2.reshape(NP)
    xs = _dispatch(x, dest)
    ys = _matmul(xs, W, bias, be2.reshape(NB), tb2.reshape(1))
    return _combine(ys, dest, Bb, Ss)


# speedup vs baseline: 1.0071x; 1.0071x over previous
import jax
import jax.numpy as jnp
from jax import lax
from jax.experimental import pallas as pl
from jax.experimental.pallas import tpu as pltpu
from jax.experimental.pallas import tpu_sc as plsc

E = 16
K = 2
BLK = 256
NP = 8192
P = NP + E * BLK
NB = P // BLK

NC = 2
NS = 16
NW = NC * NS
PER_W = NP // NW
CHUNK = 32
NPAIR = PER_W // (2 * CHUNK)


def _cumsum_axis1(x):
    n = x.shape[1]
    s = 1
    while s < n:
        x = x + jnp.concatenate(
            [jnp.zeros((x.shape[0], s), x.dtype), x[:, :-s]], axis=1)
        s *= 2
    return x


def _routing_body(idx_ref, dest_ref, be_ref, tb_ref):
    idx = idx_ref[...]
    e_iota = lax.broadcasted_iota(jnp.int32, (E, NP), 0)
    oh = (jnp.broadcast_to(idx, (E, NP)) == e_iota).astype(jnp.int32)
    cum = _cumsum_axis1(oh)
    rank = jnp.sum(jnp.where(oh == 1, cum - 1, 0), axis=0, keepdims=True)
    counts = cum[:, NP - 1:NP]
    nbk = (counts + (BLK - 1)) // BLK
    cc = nbk
    s = 1
    while s < E:
        cc = cc + jnp.concatenate(
            [jnp.zeros((s, 1), jnp.int32), cc[:-s, :]], axis=0)
        s *= 2
    blk_off = cc - nbk
    tb_ref[...] = cc[E - 1:E, :]
    dest_ref[...] = jnp.sum(
        jnp.where(oh == 1, jnp.broadcast_to(blk_off * BLK, (E, NP)), 0),
        axis=0, keepdims=True) + rank
    b_iota = lax.broadcasted_iota(jnp.int32, (E, NB), 1)
    cmp = (b_iota >= jnp.broadcast_to(blk_off, (E, NB))).astype(jnp.int32)
    be = jnp.sum(cmp, axis=0, keepdims=True) - 1
    be_ref[...] = jnp.clip(be, 0, E - 1)


def _routing(idx2):
    return pl.pallas_call(
        _routing_body,
        out_shape=(
            jax.ShapeDtypeStruct((1, NP), jnp.int32),
            jax.ShapeDtypeStruct((1, NB), jnp.int32),
            jax.ShapeDtypeStruct((1, 1), jnp.int32),
        ),
    )(idx2)


def _dispatch_body(x_hbm, dest_hbm, xs_hbm, didx0, didx1, sidx0, sidx1,
                   rows0, rows1, sg0, sg1, ss0, ss1):
    wid = lax.axis_index("s") * NC + lax.axis_index("c")
    base_w = wid * PER_W

    def load_idx(i, didx, sidx):
        base = base_w + i * CHUNK
        pltpu.sync_copy(dest_hbm.at[pl.ds(base, CHUNK)], didx)
        for v in range(CHUNK // 16):
            vec = jnp.bitwise_and(
                base + v * 16 + lax.broadcasted_iota(jnp.int32, (16,), 0),
                NP // K - 1)
            sidx[pl.ds(v * 16, 16)] = vec

    def pair(g, carry):
        @pl.when(g > 0)
        def _():
            pltpu.make_async_copy(rows0, xs_hbm.at[didx0], ss0).wait()
            pltpu.make_async_copy(rows1, xs_hbm.at[didx1], ss1).wait()

        i0 = g * 2
        load_idx(i0, didx0, sidx0)
        pltpu.async_copy(x_hbm.at[sidx0], rows0, sg0)
        load_idx(i0 + 1, didx1, sidx1)
        pltpu.async_copy(x_hbm.at[sidx1], rows1, sg1)
        pltpu.make_async_copy(x_hbm.at[sidx0], rows0, sg0).wait()
        pltpu.async_copy(rows0, xs_hbm.at[didx0], ss0)
        pltpu.make_async_copy(x_hbm.at[sidx1], rows1, sg1).wait()
        pltpu.async_copy(rows1, xs_hbm.at[didx1], ss1)
        return carry

    lax.fori_loop(0, NPAIR, pair, 0)
    pltpu.make_async_copy(rows0, xs_hbm.at[didx0], ss0).wait()
    pltpu.make_async_copy(rows1, xs_hbm.at[didx1], ss1).wait()


def _dispatch(x, dest):
    D = x.shape[1]
    mesh = plsc.VectorSubcoreMesh(core_axis_name="c", subcore_axis_name="s")
    fn = pl.kernel(
        _dispatch_body,
        out_type=jax.ShapeDtypeStruct((P, D), jnp.float32),
        mesh=mesh,
        scratch_types=[
            pltpu.VMEM((CHUNK,), jnp.int32),
            pltpu.VMEM((CHUNK,), jnp.int32),
            pltpu.VMEM((CHUNK,), jnp.int32),
            pltpu.VMEM((CHUNK,), jnp.int32),
            pltpu.VMEM((CHUNK, D), jnp.float32),
            pltpu.VMEM((CHUNK, D), jnp.float32),
            pltpu.SemaphoreType.DMA,
            pltpu.SemaphoreType.DMA,
            pltpu.SemaphoreType.DMA,
            pltpu.SemaphoreType.DMA,
        ],
    )
    return fn(x, dest)


def _matmul_body(be_ref, tb_ref, x_ref, w_ref, b_ref, o_ref):
    blk = pl.program_id(0)

    @pl.when(blk < tb_ref[0])
    def _():
        acc = lax.dot_general(
            x_ref[...], w_ref[0],
            dimension_numbers=(((1,), (1,)), ((), ())),
            preferred_element_type=jnp.float32)
        o_ref[...] = acc + b_ref[0]


def _matmul(xs, W, bias, be, tb):
    D = xs.shape[1]
    grid_spec = pltpu.PrefetchScalarGridSpec(
        num_scalar_prefetch=2,
        grid=(NB,),
        in_specs=[
            pl.BlockSpec((BLK, D),
                         lambda b, be, tb: (jnp.where(b < tb[0], b, 0), 0)),
            pl.BlockSpec((1, D, D), lambda b, be, tb: (be[b], 0, 0)),
            pl.BlockSpec((1, 1, D), lambda b, be, tb: (be[b], 0, 0)),
        ],
        out_specs=pl.BlockSpec(
            (BLK, D), lambda b, be, tb: (jnp.where(b < tb[0], b, NB - 1), 0)),
    )
    return pl.pallas_call(
        _matmul_body,
        grid_spec=grid_spec,
        out_shape=jax.ShapeDtypeStruct((P, D), jnp.float32),
    )(be, tb, xs, W, bias.reshape(E, 1, D))


def _combine_body(ys_hbm, dest_hbm, out_hbm, didx0, didx1, rows0, rows1,
                  sg0, sg1):
    wid = lax.axis_index("s") * NC + lax.axis_index("c")
    kk = wid // (NW // K)
    n0 = (wid % (NW // K)) * PER_W
    bb = n0 // (NP // K // 2)
    s_base = n0 % (NP // K // 2)

    def pair(g, carry):
        i0 = g * 2
        s0 = s_base + i0 * CHUNK
        p0 = kk * (NP // K) + n0 + i0 * CHUNK

        @pl.when(g > 0)
        def _():
            pltpu.make_async_copy(
                rows0, out_hbm.at[bb, pl.ds(s0 - 2 * CHUNK, CHUNK), kk], sg0).wait()
            pltpu.make_async_copy(
                rows1, out_hbm.at[bb, pl.ds(s0 - CHUNK, CHUNK), kk], sg1).wait()

        pltpu.sync_copy(dest_hbm.at[pl.ds(p0, CHUNK)], didx0)
        pltpu.async_copy(ys_hbm.at[didx0], rows0, sg0)
        pltpu.sync_copy(dest_hbm.at[pl.ds(p0 + CHUNK, CHUNK)], didx1)
        pltpu.async_copy(ys_hbm.at[didx1], rows1, sg1)
        pltpu.make_async_copy(ys_hbm.at[didx0], rows0, sg0).wait()
        pltpu.async_copy(rows0, out_hbm.at[bb, pl.ds(s0, CHUNK), kk], sg0)
        pltpu.make_async_copy(ys_hbm.at[didx1], rows1, sg1).wait()
        pltpu.async_copy(rows1, out_hbm.at[bb, pl.ds(s0 + CHUNK, CHUNK), kk], sg1)
        return carry

    lax.fori_loop(0, NPAIR, pair, 0)
    s_last = s_base + (NPAIR * 2 - 2) * CHUNK
    pltpu.make_async_copy(
        rows0, out_hbm.at[bb, pl.ds(s_last, CHUNK), kk], sg0).wait()
    pltpu.make_async_copy(
        rows1, out_hbm.at[bb, pl.ds(s_last + CHUNK, CHUNK), kk], sg1).wait()


def _combine(ys, dest, Bb, Ss):
    D = ys.shape[1]
    mesh = plsc.VectorSubcoreMesh(core_axis_name="c", subcore_axis_name="s")
    fn = pl.kernel(
        _combine_body,
        out_type=jax.ShapeDtypeStruct((Bb, Ss, K, D), jnp.float32),
        mesh=mesh,
        scratch_types=[
            pltpu.VMEM((CHUNK,), jnp.int32),
            pltpu.VMEM((CHUNK,), jnp.int32),
            pltpu.VMEM((CHUNK, D), jnp.float32),
            pltpu.VMEM((CHUNK, D), jnp.float32),
            pltpu.SemaphoreType.DMA,
            pltpu.SemaphoreType.DMA,
        ],
    )
    return fn(ys, dest)


def kernel(inputs, expert_indices, W, bias):
    Bb, Ss, Dd = inputs.shape
    x = inputs.reshape(Bb * Ss, Dd)
    idx2 = (expert_indices.reshape(Bb * Ss, K).T
            .reshape(1, NP).astype(jnp.int32))
    dest2, be2, tb2 = _routing(idx2)
    dest = dest2.reshape(NP)
    xs = _dispatch(x, dest)
    ys = _matmul(xs, W, bias, be2.reshape(NB), tb2.reshape(1))
    return _combine(ys, dest, Bb, Ss)
